# baseline (speedup 1.0000x reference)
import jax
import jax.numpy as jnp
from jax.experimental import pallas as pl
from jax.experimental.pallas import tpu as pltpu

_INPUT_DIM = 768
_NUM_EXPERTS = 64
_BLOCK_T = 1024
_CHUNK = 256


def _dot(a, bm):
    return jax.lax.dot_general(
        a, bm,
        dimension_numbers=(((1,), (0,)), ((), ())),
        preferred_element_type=jnp.float32,
    )


def _gating_block(x0_ref, x1_ref, x2_ref, wt_ref, b_ref, o_ref):
    w = wt_ref[...]
    logits = (_dot(x0_ref[...], w[0:_CHUNK, :])
              + _dot(x1_ref[...], w[_CHUNK:2 * _CHUNK, :])
              + _dot(x2_ref[...], w[2 * _CHUNK:3 * _CHUNK, :])
              + b_ref[...])
    v1 = jnp.max(logits, axis=1, keepdims=True)
    m1 = logits == v1
    masked = jnp.where(m1, -jnp.inf, logits)
    v2 = jnp.max(masked, axis=1, keepdims=True)
    m2 = masked == v2
    t = jnp.exp(v2 - v1)
    w1 = 1.0 / (1.0 + t)
    w2 = t * w1
    o_ref[...] = jnp.where(m1, w1, 0.0) + jnp.where(m2, w2, 0.0)


@jax.jit
def kernel(x, W, b):
    tokens = x.shape[0]
    wt = W.T
    b2 = b.reshape(1, _NUM_EXPERTS)
    grid = (tokens // _BLOCK_T,)
    xspec = lambda j: pl.BlockSpec((_BLOCK_T, _CHUNK), lambda i, j=j: (i, j))
    return pl.pallas_call(
        _gating_block,
        grid=grid,
        in_specs=[
            xspec(0),
            xspec(1),
            xspec(2),
            pl.BlockSpec((_INPUT_DIM, _NUM_EXPERTS), lambda i: (0, 0)),
            pl.BlockSpec((1, _NUM_EXPERTS), lambda i: (0, 0)),
        ],
        out_specs=pl.BlockSpec((_BLOCK_T, _NUM_EXPERTS), lambda i: (i, 0)),
        out_shape=jax.ShapeDtypeStruct((tokens, _NUM_EXPERTS), jnp.float32),
        compiler_params=pltpu.CompilerParams(
            dimension_semantics=("parallel",),
        ),
    )(x, x, x, wt, b2)


# speedup vs baseline: 1.1811x; 1.1811x over previous
import jax
import jax.numpy as jnp
from jax.experimental import pallas as pl
from jax.experimental.pallas import tpu as pltpu

_INPUT_DIM = 768
_NUM_EXPERTS = 64
_BLOCK_T = 2048
_CHUNK = 256


def _dot(a, bm):
    return jax.lax.dot_general(
        a, bm,
        dimension_numbers=(((1,), (0,)), ((), ())),
        preferred_element_type=jnp.float32,
    )


def _gating_block(x0_ref, x1_ref, x2_ref, wt_ref, b_ref, o_ref):
    w = wt_ref[...]
    logits = (_dot(x0_ref[...], w[0:_CHUNK, :])
              + _dot(x1_ref[...], w[_CHUNK:2 * _CHUNK, :])
              + _dot(x2_ref[...], w[2 * _CHUNK:3 * _CHUNK, :])
              + b_ref[...])
    v1 = jnp.max(logits, axis=1, keepdims=True)
    m1 = logits == v1
    masked = jnp.where(m1, -jnp.inf, logits)
    v2 = jnp.max(masked, axis=1, keepdims=True)
    m2 = masked == v2
    t = jnp.exp(v2 - v1)
    w1 = 1.0 / (1.0 + t)
    w2 = t * w1
    o_ref[...] = jnp.where(m1, w1, 0.0) + jnp.where(m2, w2, 0.0)


@jax.jit
def kernel(x, W, b):
    tokens = x.shape[0]
    wt = W.T
    b2 = b.reshape(1, _NUM_EXPERTS)
    grid = (tokens // _BLOCK_T,)
    xspec = lambda j: pl.BlockSpec((_BLOCK_T, _CHUNK), lambda i, j=j: (i, j))
    return pl.pallas_call(
        _gating_block,
        grid=grid,
        in_specs=[
            xspec(0),
            xspec(1),
            xspec(2),
            pl.BlockSpec((_INPUT_DIM, _NUM_EXPERTS), lambda i: (0, 0)),
            pl.BlockSpec((1, _NUM_EXPERTS), lambda i: (0, 0)),
        ],
        out_specs=pl.BlockSpec((_BLOCK_T, _NUM_EXPERTS), lambda i: (i, 0)),
        out_shape=jax.ShapeDtypeStruct((tokens, _NUM_EXPERTS), jnp.float32),
        compiler_params=pltpu.CompilerParams(
            dimension_semantics=("parallel",),
        ),
    )(x, x, x, wt, b2)


# speedup vs baseline: 1.2266x; 1.0385x over previous
import jax
import jax.numpy as jnp
from jax.experimental import pallas as pl
from jax.experimental.pallas import tpu as pltpu

_INPUT_DIM = 768
_NUM_EXPERTS = 64
_BLOCK_T = 4096
_CHUNK = 256


def _dot(a, bm):
    return jax.lax.dot_general(
        a, bm,
        dimension_numbers=(((1,), (0,)), ((), ())),
        preferred_element_type=jnp.float32,
    )


def _gating_block(x0_ref, x1_ref, x2_ref, wt_ref, b_ref, o_ref):
    w = wt_ref[...]
    logits = (_dot(x0_ref[...], w[0:_CHUNK, :])
              + _dot(x1_ref[...], w[_CHUNK:2 * _CHUNK, :])
              + _dot(x2_ref[...], w[2 * _CHUNK:3 * _CHUNK, :])
              + b_ref[...])
    v1 = jnp.max(logits, axis=1, keepdims=True)
    m1 = logits == v1
    masked = jnp.where(m1, -jnp.inf, logits)
    v2 = jnp.max(masked, axis=1, keepdims=True)
    m2 = masked == v2
    t = jnp.exp(v2 - v1)
    w1 = 1.0 / (1.0 + t)
    w2 = t * w1
    o_ref[...] = jnp.where(m1, w1, 0.0) + jnp.where(m2, w2, 0.0)


@jax.jit
def kernel(x, W, b):
    tokens = x.shape[0]
    wt = W.T
    b2 = b.reshape(1, _NUM_EXPERTS)
    grid = (tokens // _BLOCK_T,)
    xspec = lambda j: pl.BlockSpec((_BLOCK_T, _CHUNK), lambda i, j=j: (i, j))
    return pl.pallas_call(
        _gating_block,
        grid=grid,
        in_specs=[
            xspec(0),
            xspec(1),
            xspec(2),
            pl.BlockSpec((_INPUT_DIM, _NUM_EXPERTS), lambda i: (0, 0)),
            pl.BlockSpec((1, _NUM_EXPERTS), lambda i: (0, 0)),
        ],
        out_specs=pl.BlockSpec((_BLOCK_T, _NUM_EXPERTS), lambda i: (i, 0)),
        out_shape=jax.ShapeDtypeStruct((tokens, _NUM_EXPERTS), jnp.float32),
        compiler_params=pltpu.CompilerParams(
            dimension_semantics=("parallel",),
        ),
    )(x, x, x, wt, b2)
